# Initial kernel scaffold; baseline (speedup 1.0000x reference)
import jax
import jax.numpy as jnp
from jax.experimental import pallas as pl


def kernel(x):
    raise NotImplementedError("write your pallas kernel here")


# speedup vs baseline: 4.3917x; 4.3917x over previous
import functools

import jax
import jax.numpy as jnp
from jax.experimental import pallas as pl
from jax.experimental.pallas import tpu as pltpu

_K = 16
_NEG = float(jnp.finfo(jnp.float32).min)
_BIG = 2**30


def _normalize_body(x_ref, o_ref):
    x = x_ref[...]
    n = jnp.sqrt(jnp.sum(x * x, axis=1, keepdims=True))
    o_ref[...] = x / jnp.maximum(n, 1e-12)


def _topk_body(xr_ref, xall_ref, o_ref, cand_v, cand_i, *, n, blk_c, k):
    r = xr_ref.shape[0]
    nc = n // blk_c
    xr = xr_ref[...]
    for c in range(nc):
        xc = xall_ref[pl.ds(c * blk_c, blk_c), :]
        s = jax.lax.dot_general(xr, xc, (((1,), (1,)), ((), ())),
                                preferred_element_type=jnp.float32)
        col = jax.lax.broadcasted_iota(jnp.int32, (r, blk_c), 1) + jnp.int32(c * blk_c)
        for j in range(k):
            m = jnp.max(s, axis=1, keepdims=True)
            idx = jnp.min(jnp.where(s == m, col, _BIG), axis=1, keepdims=True)
            cand_v[:, c * k + j] = m[:, 0]
            cand_i[:, c * k + j] = idx[:, 0]
            s = jnp.where(col == idx, _NEG, s)
    v = cand_v[...]
    ci = cand_i[...]
    for j in range(k):
        m = jnp.max(v, axis=1, keepdims=True)
        idx = jnp.min(jnp.where(v == m, ci, _BIG), axis=1, keepdims=True)
        o_ref[:, j] = idx[:, 0]
        v = jnp.where(ci == idx, _NEG, v)


def _knn(x, *, blk_r, blk_c, interpret=False):
    n, d = x.shape
    xn = pl.pallas_call(
        _normalize_body,
        grid=(n // blk_r,),
        in_specs=[pl.BlockSpec((blk_r, d), lambda i: (i, 0))],
        out_specs=pl.BlockSpec((blk_r, d), lambda i: (i, 0)),
        out_shape=jax.ShapeDtypeStruct((n, d), jnp.float32),
        interpret=interpret,
    )(x)

    nc = n // blk_c
    body = functools.partial(_topk_body, n=n, blk_c=blk_c, k=_K)
    nn_idx = pl.pallas_call(
        body,
        grid=(n // blk_r,),
        in_specs=[
            pl.BlockSpec((blk_r, d), lambda i: (i, 0)),
            pl.BlockSpec((n, d), lambda i: (0, 0)),
        ],
        out_specs=pl.BlockSpec((blk_r, _K), lambda i: (i, 0)),
        out_shape=jax.ShapeDtypeStruct((n, _K), jnp.int32),
        scratch_shapes=[
            pltpu.VMEM((blk_r, nc * _K), jnp.float32),
            pltpu.VMEM((blk_r, nc * _K), jnp.int32),
        ],
        interpret=interpret,
    )(xn, xn)
    return nn_idx


def kernel(x):
    n = x.shape[0]
    xvec = x.reshape(n, -1)
    return _knn(xvec, blk_r=256, blk_c=1024)


# speedup vs baseline: 4.3920x; 1.0001x over previous
import functools

import jax
import jax.numpy as jnp
from jax.experimental import pallas as pl
from jax.experimental.pallas import tpu as pltpu

_K = 16
_NEG = float(jnp.finfo(jnp.float32).min)
_BIG = 2**30


def _normalize_body(x_ref, o_ref):
    x = x_ref[...]
    n = jnp.sqrt(jnp.sum(x * x, axis=1, keepdims=True))
    o_ref[...] = x / jnp.maximum(n, 1e-12)


def _topk_body(xr_ref, xall_ref, o_ref, cand_v, cand_i, *, n, blk_c, k):
    r = xr_ref.shape[0]
    nc = n // blk_c
    xr = xr_ref[...]

    def dot_block(c):
        xc = xall_ref[pl.ds(c * blk_c, blk_c), :]
        return jax.lax.dot_general(xr, xc, (((1,), (1,)), ((), ())),
                                   preferred_element_type=jnp.float32)

    s_next = dot_block(0)
    for c in range(nc):
        s = s_next
        if c + 1 < nc:
            s_next = dot_block(c + 1)
        col = jax.lax.broadcasted_iota(jnp.int32, (r, blk_c), 1) + jnp.int32(c * blk_c)
        for j in range(k):
            m = jnp.max(s, axis=1, keepdims=True)
            idx = jnp.min(jnp.where(s == m, col, _BIG), axis=1, keepdims=True)
            cand_v[:, c * k + j] = m[:, 0]
            cand_i[:, c * k + j] = idx[:, 0]
            s = jnp.where(col == idx, _NEG, s)
    v = cand_v[...]
    ci = cand_i[...]
    for j in range(k):
        m = jnp.max(v, axis=1, keepdims=True)
        idx = jnp.min(jnp.where(v == m, ci, _BIG), axis=1, keepdims=True)
        o_ref[:, j] = idx[:, 0]
        v = jnp.where(ci == idx, _NEG, v)


def _knn(x, *, blk_r, blk_c, interpret=False):
    n, d = x.shape
    xn = pl.pallas_call(
        _normalize_body,
        grid=(n // blk_r,),
        in_specs=[pl.BlockSpec((blk_r, d), lambda i: (i, 0))],
        out_specs=pl.BlockSpec((blk_r, d), lambda i: (i, 0)),
        out_shape=jax.ShapeDtypeStruct((n, d), jnp.float32),
        interpret=interpret,
    )(x)

    nc = n // blk_c
    body = functools.partial(_topk_body, n=n, blk_c=blk_c, k=_K)
    nn_idx = pl.pallas_call(
        body,
        grid=(n // blk_r,),
        in_specs=[
            pl.BlockSpec((blk_r, d), lambda i: (i, 0)),
            pl.BlockSpec((n, d), lambda i: (0, 0)),
        ],
        out_specs=pl.BlockSpec((blk_r, _K), lambda i: (i, 0)),
        out_shape=jax.ShapeDtypeStruct((n, _K), jnp.int32),
        scratch_shapes=[
            pltpu.VMEM((blk_r, nc * _K), jnp.float32),
            pltpu.VMEM((blk_r, nc * _K), jnp.int32),
        ],
        interpret=interpret,
    )(xn, xn)
    return nn_idx


def kernel(x):
    n = x.shape[0]
    xvec = x.reshape(n, -1)
    return _knn(xvec, blk_r=256, blk_c=1024)
